# Initial kernel scaffold; baseline (speedup 1.0000x reference)
import jax
import jax.numpy as jnp
from jax.experimental import pallas as pl


def kernel(x, edge_index, batch, bn_w, bn_b, W3, att_src, att_dst, gat_b, conv_w, conv_b):
    raise NotImplementedError("write your pallas kernel here")


# speedup vs baseline: 5.0637x; 5.0637x over previous
import functools

import jax
import jax.numpy as jnp
from jax import lax
from jax.experimental import pallas as pl
from jax.experimental.pallas import tpu as pltpu

N = 23808
E = 380928
F_IN = 128
H = 4
OUT = 24
B = 128
NPG = 186
KW = 62
P = NPG - KW + 1

_HIGH = lax.Precision.HIGHEST


def _lrelu(x, slope):
    return jnp.where(x > 0, x, slope * x)


_PBN = 2976


def _stats_body(x_ref, stat_ref):
    i = pl.program_id(0)
    x = x_ref[...]
    blk = jnp.concatenate([jnp.sum(x, axis=0, keepdims=True),
                           jnp.sum(x * x, axis=0, keepdims=True)], axis=0)

    @pl.when(i == 0)
    def _():
        stat_ref[...] = blk

    @pl.when(i > 0)
    def _():
        stat_ref[...] += blk


def _prep_main_body(x_ref, stat_ref, bnw_ref, bnb_ref, w3t_ref, aw_src_ref,
                    aw_dst_ref, xwp_ref, asrc_ref, adst_ref, m_ref):
    i = pl.program_id(0)
    stat = stat_ref[...]
    mean = stat[0:1] / N
    var = stat[1:2] / N - mean * mean
    rstd = lax.rsqrt(var + 1e-5)
    xh = (x_ref[...] - mean) * (rstd * bnw_ref[...]) + bnb_ref[...]
    xw = jnp.dot(xh, w3t_ref[...], preferred_element_type=jnp.float32,
                 precision=_HIGH)
    a_src = jnp.dot(xw, aw_src_ref[...], preferred_element_type=jnp.float32,
                    precision=_HIGH)
    a_dst = jnp.dot(xw, aw_dst_ref[...], preferred_element_type=jnp.float32,
                    precision=_HIGH)
    xwp_ref[0] = xw[:, : 2 * OUT]
    xwp_ref[1] = xw[:, 2 * OUT:]
    asrc_ref[0] = a_src[:, 0:2]
    asrc_ref[1] = a_src[:, 2:4]
    adst_ref[0] = a_dst[:, 0:2]
    adst_ref[1] = a_dst[:, 2:4]
    ms = jnp.max(a_src, axis=0)
    md = jnp.max(a_dst, axis=0)
    row = jnp.pad(jnp.concatenate([ms, jnp.zeros(4, jnp.float32), md]),
                  (0, 116))[None, :]

    @pl.when(i == 0)
    def _():
        m_ref[...] = row

    @pl.when(i > 0)
    def _():
        m_ref[...] = jnp.maximum(m_ref[...], row)


def _prep(x, bn_w, bn_b, w3t, aw_src, aw_dst):
    nb = N // _PBN
    stat = pl.pallas_call(
        _stats_body,
        grid=(nb,),
        in_specs=[pl.BlockSpec((_PBN, F_IN), lambda i: (i, 0))],
        out_specs=pl.BlockSpec((2, F_IN), lambda i: (0, 0)),
        out_shape=jax.ShapeDtypeStruct((2, F_IN), jnp.float32),
    )(x)
    return pl.pallas_call(
        _prep_main_body,
        grid=(nb,),
        in_specs=[
            pl.BlockSpec((_PBN, F_IN), lambda i: (i, 0)),
            pl.BlockSpec((2, F_IN), lambda i: (0, 0)),
            pl.BlockSpec((1, F_IN), lambda i: (0, 0)),
            pl.BlockSpec((1, F_IN), lambda i: (0, 0)),
            pl.BlockSpec((F_IN, H * OUT), lambda i: (0, 0)),
            pl.BlockSpec((H * OUT, H), lambda i: (0, 0)),
            pl.BlockSpec((H * OUT, H), lambda i: (0, 0)),
        ],
        out_specs=(
            pl.BlockSpec((2, _PBN, 2 * OUT), lambda i: (0, i, 0)),
            pl.BlockSpec((2, _PBN, 2), lambda i: (0, i, 0)),
            pl.BlockSpec((2, _PBN, 2), lambda i: (0, i, 0)),
            pl.BlockSpec((1, F_IN), lambda i: (0, 0)),
        ),
        out_shape=(
            jax.ShapeDtypeStruct((2, N, 2 * OUT), jnp.float32),
            jax.ShapeDtypeStruct((2, N, 2), jnp.float32),
            jax.ShapeDtypeStruct((2, N, 2), jnp.float32),
            jax.ShapeDtypeStruct((1, F_IN), jnp.float32),
        ),
    )(x, stat, bn_w[None, :], bn_b[None, :], w3t, aw_src, aw_dst)


_BN = 2976


def _norm_body(acc_ref, xwp_ref, asrc_ref, adst_ref, m_ref, gatb_ref, hn_ref):
    asrc = jnp.concatenate([asrc_ref[0], asrc_ref[1]], axis=1)
    adst = jnp.concatenate([adst_ref[0], adst_ref[1]], axis=1)
    m = m_ref[0:1, 0:4] + m_ref[0:1, 8:12]
    e_self = jnp.exp(_lrelu(asrc + adst, 0.2) - m)
    xw = jnp.concatenate([xwp_ref[0], xwp_ref[1]], axis=1)
    s4 = jnp.concatenate(
        [acc_ref[0, :, 48:50], acc_ref[1, :, 48:50]], axis=1) + e_self
    num = jnp.concatenate([acc_ref[0, :, :48], acc_ref[1, :, :48]], axis=1)
    num = num + xw * jnp.repeat(e_self, OUT, axis=1)
    hn = num.reshape(_BN, H, OUT) / s4[:, :, None]
    hn = jnp.mean(hn, axis=1) + gatb_ref[...]
    hn_ref[...] = jnp.where(hn > 0, hn, jnp.exp(jnp.minimum(hn, 0.0)) - 1.0)


def _gat_normalize(acc, xwp, asrc, adst, m8, gat_b):
    nb = N // _BN
    return pl.pallas_call(
        _norm_body,
        grid=(nb,),
        in_specs=[
            pl.BlockSpec((2, _BN, 64), lambda i: (0, i, 0)),
            pl.BlockSpec((2, _BN, 2 * OUT), lambda i: (0, i, 0)),
            pl.BlockSpec((2, _BN, 2), lambda i: (0, i, 0)),
            pl.BlockSpec((2, _BN, 2), lambda i: (0, i, 0)),
            pl.BlockSpec((1, F_IN), lambda i: (0, 0)),
            pl.BlockSpec((1, OUT), lambda i: (0, 0)),
        ],
        out_specs=pl.BlockSpec((_BN, OUT), lambda i: (i, 0)),
        out_shape=jax.ShapeDtypeStruct((N, OUT), jnp.float32),
    )(acc, xwp, asrc, adst, m8, gat_b[None, :])


_CB = 16


def _conv_body(hg_ref, convwt_ref, convb_ref, out_ref, y_ref):
    y_ref[...] = jnp.zeros_like(y_ref)

    def step(k, carry):
        hk = hg_ref[:, pl.ds(k, P), :].reshape(_CB * P, OUT)
        wk = convwt_ref[k]
        y_ref[...] += jnp.dot(hk, wk, preferred_element_type=jnp.float32,
                              precision=_HIGH)
        return carry

    lax.fori_loop(0, KW, step, 0)
    y = y_ref[...].reshape(_CB, P, 8) + convb_ref[...]
    y = jnp.transpose(y, (0, 2, 1))
    out_ref[...] = _lrelu(y, 0.01)


def _conv(hg, convw_t, conv_b):
    return pl.pallas_call(
        _conv_body,
        grid=(B // _CB,),
        in_specs=[
            pl.BlockSpec((_CB, NPG, OUT), lambda i: (i, 0, 0)),
            pl.BlockSpec((KW, OUT, 8), lambda i: (0, 0, 0)),
            pl.BlockSpec((1, 1, 8), lambda i: (0, 0, 0)),
        ],
        out_specs=pl.BlockSpec((_CB, 8, P), lambda i: (i, 0, 0)),
        out_shape=jax.ShapeDtypeStruct((B, 8, P), jnp.float32),
        scratch_shapes=[pltpu.VMEM((_CB * P, 8), jnp.float32)],
    )(hg, convw_t, conv_b[None, None, :])


def _edge_phase_jax(xwp, asrc_p, adst_p, m8, src, dst):
    m = m8[0, :4] + m8[0, 8:12]
    asrc = jnp.concatenate([asrc_p[0], asrc_p[1]], axis=1)
    adst = jnp.concatenate([adst_p[0], adst_p[1]], axis=1)
    al = asrc[src] + adst[dst]
    e = jnp.exp(_lrelu(al, 0.2) - m[None, :])
    accs = []
    for c in (0, 1):
        rows = xwp[c][src]
        pay = jnp.concatenate(
            [e[:, 2 * c:2 * c + 1] * rows[:, :OUT],
             e[:, 2 * c + 1:2 * c + 2] * rows[:, OUT:],
             e[:, 2 * c:2 * c + 2],
             jnp.zeros((E, 14), jnp.float32)], axis=1)
        accs.append(jax.ops.segment_sum(pay, dst, num_segments=N))
    return jnp.stack(accs)


def kernel(x, edge_index, batch, bn_w, bn_b, W3, att_src, att_dst, gat_b,
           conv_w, conv_b):
    w3t = W3.T
    eye = jnp.eye(H, dtype=jnp.float32)
    aw_src = (att_src[0][:, :, None] * eye[:, None, :]).reshape(H * OUT, H)
    aw_dst = (att_dst[0][:, :, None] * eye[:, None, :]).reshape(H * OUT, H)
    convw_t = jnp.transpose(conv_w, (2, 1, 0))

    xwp, asrc_p, adst_p, m8 = _prep(x, bn_w, bn_b, w3t, aw_src, aw_dst)
    acc = _edge_phase_jax(xwp, asrc_p, adst_p, m8,
                          edge_index[0], edge_index[1])
    hn = _gat_normalize(acc, xwp, asrc_p, adst_p, m8, gat_b)
    return _conv(hn.reshape(B, NPG, OUT), convw_t, conv_b)
